# baseline (speedup 1.0000x reference)
import jax
import jax.numpy as jnp
from jax import lax
from jax.experimental import pallas as pl
from jax.experimental.pallas import tpu as pltpu
from jax.experimental.pallas import tpu_sc as plsc

VOCAB = 100000
SEQ = 200
DIM = 128
BATCH = 1024

_INFO = plsc.get_sparse_core_info()
_NC = _INFO.num_cores
_NS = _INFO.num_subcores
_NW = _NC * _NS
_ROWS_PER_W = BATCH // _NW
_PER_W = _ROWS_PER_W * SEQ

_CHUNKS = ((0, 104), (104, 96))

_LANES = 16
_VECS_PER_LINE = DIM // _LANES
_NBUF = 3


def _body(x_hbm, tok_hbm, pos_hbm, out_hbm,
          idx_v, pos_v, buf0, buf1, buf2,
          g0a, g0b, g1a, g1b, g2a, g2b, s0, s1, s2):
    bufs = (buf0, buf1, buf2)
    gsems = ((g0a, g0b), (g1a, g1b), (g2a, g2b))
    ssems = (s0, s1, s2)

    wid = lax.axis_index("s") * _NC + lax.axis_index("c")
    row0 = wid * _ROWS_PER_W
    base0 = row0 * SEQ

    pltpu.sync_copy(pos_hbm, pos_v)
    pltpu.sync_copy(x_hbm.at[pl.ds(base0, _PER_W)], idx_v)

    def gather(r):
        b = r % _NBUF
        for ci, (off, n) in enumerate(_CHUNKS):
            pltpu.async_copy(
                tok_hbm.at[idx_v.at[pl.ds(r * SEQ + off, n)]],
                bufs[b].at[pl.ds(off, n), :],
                gsems[b][ci],
            )

    def wait_gather_chunk(r, ci, off, n):
        b = r % _NBUF
        pltpu.make_async_copy(
            tok_hbm.at[idx_v.at[pl.ds(r * SEQ + off, n)]],
            bufs[b].at[pl.ds(off, n), :],
            gsems[b][ci],
        ).wait()

    def store_chunk(r, off, n):
        b = r % _NBUF
        pltpu.async_copy(
            bufs[b].at[pl.ds(off, n), :],
            out_hbm.at[pl.ds(base0 + r * SEQ + off, n)],
            ssems[b],
        )

    def wait_store(r):
        b = r % _NBUF
        for off, n in _CHUNKS:
            pltpu.make_async_copy(
                bufs[b].at[pl.ds(off, n), :],
                out_hbm.at[pl.ds(base0 + r * SEQ + off, n)],
                ssems[b],
            ).wait()

    gather(0)
    for r in range(_ROWS_PER_W):
        if r + 1 < _ROWS_PER_W:
            if r >= 2:
                wait_store(r - 2)
            gather(r + 1)
        buf = bufs[r % _NBUF]
        for ci, (off, n) in enumerate(_CHUNKS):
            wait_gather_chunk(r, ci, off, n)

            @pl.loop(0, n)
            def _line(l):
                for j in range(_VECS_PER_LINE):
                    sl = pl.ds(j * _LANES, _LANES)
                    buf[off + l, sl] = buf[off + l, sl] + pos_v[off + l, sl]

            store_chunk(r, off, n)
    for r in range(_ROWS_PER_W - 3, _ROWS_PER_W):
        wait_store(r)


@jax.jit
def _run(x_flat, token_table, position_table):
    mesh = plsc.VectorSubcoreMesh(core_axis_name="c", subcore_axis_name="s")
    return pl.kernel(
        _body,
        out_type=jax.ShapeDtypeStruct((BATCH * SEQ, DIM), jnp.float32),
        mesh=mesh,
        scratch_types=(
            [pltpu.VMEM((_PER_W,), jnp.int32),
             pltpu.VMEM((SEQ, DIM), jnp.float32)]
            + [pltpu.VMEM((SEQ, DIM), jnp.float32)] * _NBUF
            + [pltpu.SemaphoreType.DMA] * (3 * _NBUF)
        ),
    )(x_flat, token_table, position_table)


def kernel(x, token_table, position_table):
    x_flat = x.reshape(-1).astype(jnp.int32)
    out = _run(x_flat, token_table, position_table)
    return out.reshape(x.shape[0], x.shape[1], DIM)


# speedup vs baseline: 1.0118x; 1.0118x over previous
import jax
import jax.numpy as jnp
from jax import lax
from jax.experimental import pallas as pl
from jax.experimental.pallas import tpu as pltpu
from jax.experimental.pallas import tpu_sc as plsc

VOCAB = 100000
SEQ = 200
DIM = 128
BATCH = 1024

_INFO = plsc.get_sparse_core_info()
_NC = _INFO.num_cores
_NS = _INFO.num_subcores
_NW = _NC * _NS
_ROWS_PER_W = BATCH // _NW

_CHUNKS = ((0, 104), (104, 96))

_LANES = 16
_VECS_PER_LINE = DIM // _LANES
_NBUF = 3


def _body(x_hbm, tok_hbm, pos_hbm, out_hbm,
          idx_v, pos_v, buf0, buf1, buf2,
          g0, g1, g2, s0, s1, s2):
    bufs = (buf0, buf1, buf2)
    gsems = (g0, g1, g2)
    ssems = (s0, s1, s2)

    wid = lax.axis_index("s") * _NC + lax.axis_index("c")
    row0 = wid * _ROWS_PER_W
    base0 = row0 * SEQ

    pltpu.sync_copy(pos_hbm, pos_v)
    pltpu.sync_copy(x_hbm.at[pl.ds(base0, _ROWS_PER_W * SEQ)], idx_v)

    def gather(r):
        b = r % _NBUF
        for off, n in _CHUNKS:
            pltpu.async_copy(
                tok_hbm.at[idx_v.at[pl.ds(r * SEQ + off, n)]],
                bufs[b].at[pl.ds(off, n), :],
                gsems[b],
            )

    def wait_gather(r):
        b = r % _NBUF
        for off, n in _CHUNKS:
            pltpu.make_async_copy(
                tok_hbm.at[idx_v.at[pl.ds(r * SEQ + off, n)]],
                bufs[b].at[pl.ds(off, n), :],
                gsems[b],
            ).wait()

    def store(r):
        b = r % _NBUF
        pltpu.async_copy(bufs[b], out_hbm.at[pl.ds(base0 + r * SEQ, SEQ)],
                         ssems[b])

    def wait_store(r):
        b = r % _NBUF
        pltpu.make_async_copy(bufs[b],
                              out_hbm.at[pl.ds(base0 + r * SEQ, SEQ)],
                              ssems[b]).wait()

    gather(0)
    for r in range(_ROWS_PER_W):
        if r + 1 < _ROWS_PER_W:
            if r >= 2:
                wait_store(r - 2)
            gather(r + 1)
        wait_gather(r)
        buf = bufs[r % _NBUF]

        @pl.loop(0, SEQ)
        def _line(i):
            for j in range(_VECS_PER_LINE):
                sl = pl.ds(j * _LANES, _LANES)
                buf[i, sl] = buf[i, sl] + pos_v[i, sl]

        store(r)
    for r in range(_ROWS_PER_W - 3, _ROWS_PER_W):
        wait_store(r)


@jax.jit
def _run(x_flat, token_table, position_table):
    mesh = plsc.VectorSubcoreMesh(core_axis_name="c", subcore_axis_name="s")
    return pl.kernel(
        _body,
        out_type=jax.ShapeDtypeStruct((BATCH * SEQ, DIM), jnp.float32),
        mesh=mesh,
        scratch_types=[
            pltpu.VMEM((_ROWS_PER_W * SEQ,), jnp.int32),
            pltpu.VMEM((SEQ, DIM), jnp.float32),
            pltpu.VMEM((SEQ, DIM), jnp.float32),
            pltpu.VMEM((SEQ, DIM), jnp.float32),
            pltpu.VMEM((SEQ, DIM), jnp.float32),
            pltpu.SemaphoreType.DMA,
            pltpu.SemaphoreType.DMA,
            pltpu.SemaphoreType.DMA,
            pltpu.SemaphoreType.DMA,
            pltpu.SemaphoreType.DMA,
            pltpu.SemaphoreType.DMA,
        ],
    )(x_flat, token_table, position_table)


def kernel(x, token_table, position_table):
    x_flat = x.reshape(-1).astype(jnp.int32)
    out = _run(x_flat, token_table, position_table)
    return out.reshape(x.shape[0], x.shape[1], DIM)


# speedup vs baseline: 1.0144x; 1.0025x over previous
import jax
import jax.numpy as jnp
from jax import lax
from jax.experimental import pallas as pl
from jax.experimental.pallas import tpu as pltpu
from jax.experimental.pallas import tpu_sc as plsc

VOCAB = 100000
SEQ = 200
DIM = 128
BATCH = 1024

_INFO = plsc.get_sparse_core_info()
_NC = _INFO.num_cores
_NS = _INFO.num_subcores
_NW = _NC * _NS
_ROWS_PER_W = BATCH // _NW

_CHUNKS = ((0, 104), (104, 96))

_LANES = 16
_VECS_PER_LINE = DIM // _LANES
_NBUF = 3


def _body(x_hbm, tok_hbm, pos_hbm, out_hbm,
          idx_v, pos_v, buf0, buf1, buf2,
          g0, g1, g2, s0, s1, s2):
    bufs = (buf0, buf1, buf2)
    gsems = (g0, g1, g2)
    ssems = (s0, s1, s2)

    wid = lax.axis_index("s") * _NC + lax.axis_index("c")
    row0 = wid * _ROWS_PER_W
    base0 = row0 * SEQ

    pltpu.sync_copy(pos_hbm, pos_v)
    pltpu.sync_copy(x_hbm.at[pl.ds(base0, _ROWS_PER_W * SEQ)], idx_v)

    def gather(r):
        b = r % _NBUF
        for off, n in _CHUNKS:
            pltpu.async_copy(
                tok_hbm.at[idx_v.at[pl.ds(r * SEQ + off, n)]],
                bufs[b].at[pl.ds(off, n), :],
                gsems[b],
            )

    def wait_gather(r):
        b = r % _NBUF
        for off, n in _CHUNKS:
            pltpu.make_async_copy(
                tok_hbm.at[idx_v.at[pl.ds(r * SEQ + off, n)]],
                bufs[b].at[pl.ds(off, n), :],
                gsems[b],
            ).wait()

    def store(r):
        b = r % _NBUF
        pltpu.async_copy(bufs[b], out_hbm.at[row0 + r], ssems[b])

    def wait_store(r):
        b = r % _NBUF
        pltpu.make_async_copy(bufs[b], out_hbm.at[row0 + r],
                              ssems[b]).wait()

    gather(0)
    for r in range(_ROWS_PER_W):
        if r + 1 < _ROWS_PER_W:
            if r >= 2:
                wait_store(r - 2)
            gather(r + 1)
        wait_gather(r)
        buf = bufs[r % _NBUF]

        @pl.loop(0, SEQ)
        def _line(i):
            for j in range(_VECS_PER_LINE):
                sl = pl.ds(j * _LANES, _LANES)
                buf[i, sl] = buf[i, sl] + pos_v[i, sl]

        store(r)
    for r in range(_ROWS_PER_W - 3, _ROWS_PER_W):
        wait_store(r)


@jax.jit
def _run(x_flat, token_table, position_table):
    mesh = plsc.VectorSubcoreMesh(core_axis_name="c", subcore_axis_name="s")
    return pl.kernel(
        _body,
        out_type=jax.ShapeDtypeStruct((BATCH, SEQ, DIM), jnp.float32),
        mesh=mesh,
        scratch_types=[
            pltpu.VMEM((_ROWS_PER_W * SEQ,), jnp.int32),
            pltpu.VMEM((SEQ, DIM), jnp.float32),
            pltpu.VMEM((SEQ, DIM), jnp.float32),
            pltpu.VMEM((SEQ, DIM), jnp.float32),
            pltpu.VMEM((SEQ, DIM), jnp.float32),
            pltpu.SemaphoreType.DMA,
            pltpu.SemaphoreType.DMA,
            pltpu.SemaphoreType.DMA,
            pltpu.SemaphoreType.DMA,
            pltpu.SemaphoreType.DMA,
            pltpu.SemaphoreType.DMA,
        ],
    )(x_flat, token_table, position_table)


def kernel(x, token_table, position_table):
    x_flat = x.reshape(-1).astype(jnp.int32)
    return _run(x_flat, token_table, position_table)
